# Initial kernel scaffold; baseline (speedup 1.0000x reference)
import jax
import jax.numpy as jnp
from jax.experimental import pallas as pl


def kernel(pokemon_state, move_state, type_state, ability_state, item_state, fieldeffect_state, pokemon_table, move_table, type_table, ability_table, item_table, fieldeffect_tables):
    raise NotImplementedError("write your pallas kernel here")


# speedup vs baseline: 5.3189x; 5.3189x over previous
import functools

import jax
import jax.numpy as jnp
from jax import lax
from jax.experimental import pallas as pl
from jax.experimental.pallas import tpu as pltpu
from jax.experimental.pallas import tpu_sc as plsc

_MAX_NORM = 1.0

_B = 4096
_SUBROWS = 407
_NW = 32
_STREAMS_W = _SUBROWS * (_B // _NW) // 128
_G = 11
_NGROUPS = _STREAMS_W // _G


def _normalize_tables_tc(*tables):
    def body(*refs):
        n = len(refs) // 2
        for src, dst in zip(refs[:n], refs[n:]):
            x = src[...]
            nrm = jnp.sqrt(jnp.sum(x * x, axis=-1, keepdims=True))
            scale = jnp.where(nrm > _MAX_NORM,
                              _MAX_NORM / jnp.maximum(nrm, 1e-12), 1.0)
            dst[...] = x * scale

    out_shapes = [jax.ShapeDtypeStruct(t.shape, t.dtype) for t in tables]
    return pl.pallas_call(body, out_shape=out_shapes)(*tables)


def _sc_gather(u, idx2d):
    mesh = plsc.VectorSubcoreMesh(core_axis_name="c", subcore_axis_name="s")

    @functools.partial(
        pl.kernel,
        mesh=mesh,
        compiler_params=pltpu.CompilerParams(use_tc_tiling_on_sc=False),
        out_type=jax.ShapeDtypeStruct((_B * _SUBROWS, 16), jnp.float32),
        scratch_types=[
            pltpu.VMEM((_STREAMS_W, 128), jnp.int32),
            pltpu.VMEM((_G * 128, 16), jnp.float32),
            pltpu.SemaphoreType.DMA,
        ],
    )
    def k(u_hbm, idx_hbm, out_hbm, idx_v, buf_v, sem):
        wid = lax.axis_index("s") * 2 + lax.axis_index("c")
        pltpu.sync_copy(idx_hbm.at[wid], idx_v)

        def group(g, carry):
            base = g * _G
            copies = [
                pltpu.async_copy(u_hbm.at[idx_v.at[base + j]],
                                 buf_v.at[pl.ds(j * 128, 128)], sem)
                for j in range(_G)
            ]
            for c in copies:
                c.wait()
            out0 = wid * (_STREAMS_W * 128) + g * (_G * 128)
            pltpu.sync_copy(buf_v, out_hbm.at[pl.ds(out0, _G * 128)])
            return carry

        lax.fori_loop(0, _NGROUPS, group, 0)

    return k(u, idx2d)


def kernel(pokemon_state, move_state, type_state, ability_state, item_state,
           fieldeffect_state, pokemon_table, move_table, type_table,
           ability_table, item_table, fieldeffect_tables):
    B = pokemon_state.shape[0]
    pt, mt, tt, at_, it, ft = _normalize_tables_tc(
        pokemon_table, move_table, type_table, ability_table, item_table,
        fieldeffect_tables.reshape(46, 16))

    u = jnp.concatenate([
        pt.reshape(-1, 16), mt.reshape(-1, 16), tt.reshape(-1, 16),
        at_.reshape(-1, 16), it.reshape(-1, 16), ft,
    ], axis=0)

    k4 = jnp.arange(4, dtype=jnp.int32)
    k2 = jnp.arange(2, dtype=jnp.int32)
    p_i = (pokemon_state[..., None] * 4 + k4).reshape(B, 48)
    m_i = (move_state[..., None] * 4 + k4).reshape(B, 192) + 4096
    t_i = (type_state[..., None] * 2 + k2).reshape(B, 48) + 8192
    a_i = (ability_state[..., None] * 4 + k4).reshape(B, 48) + 8232
    i_i = (item_state[..., None] * 4 + k4).reshape(B, 48) + 9432
    f_i = (jnp.arange(23, dtype=jnp.int32) * 2 + fieldeffect_state) + 13528
    flat = jnp.concatenate([p_i, m_i, t_i, a_i, i_i, f_i], axis=1)
    idx3d = flat.astype(jnp.int32).reshape(_NW, _STREAMS_W, 128)

    out = _sc_gather(u, idx3d)
    return out.reshape(B, _SUBROWS * 16)
